# Initial kernel scaffold; baseline (speedup 1.0000x reference)
import jax
import jax.numpy as jnp
from jax.experimental import pallas as pl


def kernel(pc, feats, W, b):
    raise NotImplementedError("write your pallas kernel here")


# speedup vs baseline: 10.7539x; 10.7539x over previous
import functools

import jax
import jax.numpy as jnp
from jax.experimental import pallas as pl


def _locse_block(pcq_ref, pcT_ref, pcrows_ref, feats_ref, wg_ref, a8_ref,
                 v_ref, b_ref, out_ref, *, k1, n, bq):
    q = pcq_ref[0]
    pT = pcT_ref[0]
    prow = pcrows_ref[0]
    feats = feats_ref[0]

    dot = jnp.dot(q, pT, preferred_element_type=jnp.float32)
    sqq = jnp.sum(q * q, axis=1, keepdims=True)
    sqp = jnp.sum(pT * pT, axis=0, keepdims=True)
    d2 = (sqq + sqp) - 2.0 * dot

    iota = jax.lax.broadcasted_iota(jnp.int32, (bq, n), 1)
    bigi = jnp.int32(n)
    inf = jnp.float32(jnp.inf)

    gpts = []
    d2m = d2
    for k in range(k1):
        minv = jnp.min(d2m, axis=1, keepdims=True)
        idx = jnp.min(jnp.where(d2m == minv, iota, bigi), axis=1,
                      keepdims=True)
        onehot = (iota == idx)
        d2m = jnp.where(onehot, inf, d2m)
        oh_f = onehot.astype(jnp.float32)
        p_k = jnp.dot(oh_f, prow, preferred_element_type=jnp.float32)
        f_k = jnp.dot(oh_f, feats, preferred_element_type=jnp.float32)
        gpts.append(p_k)
        out_ref[0, :, k, 64:128] = f_k

    g = jnp.concatenate(gpts, axis=1)
    t = jnp.dot(g, wg_ref[...], preferred_element_type=jnp.float32)
    tb = t + b_ref[...]
    for k in range(k1):
        p_k = gpts[k]
        e_k = jnp.dot(p_k, a8_ref[...], preferred_element_type=jnp.float32)
        norm_k = jnp.sqrt(jnp.sum(p_k * p_k, axis=1, keepdims=True))
        r_k = jnp.maximum(e_k + norm_k * v_ref[...] + tb, 0.0)
        out_ref[0, :, k, 0:64] = r_k


@jax.jit
def kernel(pc, feats, W, b):
    B, N, DIMS = pc.shape
    U = feats.shape[-1]
    K1 = (W.shape[0] - DIMS) // (DIMS + 1)
    BQ = 128

    pc8 = jnp.pad(pc, ((0, 0), (0, 0), (0, 8 - DIMS)))
    pcT = jnp.transpose(pc8, (0, 2, 1))

    w_xyz = jnp.stack([W[DIMS + (DIMS + 1) * k: 2 * DIMS + (DIMS + 1) * k]
                       for k in range(K1)])
    wg = jnp.pad(w_xyz, ((0, 0), (0, 8 - DIMS), (0, 0))).reshape(8 * K1, U)
    a8 = jnp.pad(W[0:DIMS] - jnp.sum(w_xyz, axis=0), ((0, 8 - DIMS), (0, 0)))
    v = jnp.sum(jnp.stack([W[2 * DIMS + (DIMS + 1) * k] for k in range(K1)]),
                axis=0, keepdims=True)
    bb = b.reshape(1, U)

    grid = (B, N // BQ)
    out = pl.pallas_call(
        functools.partial(_locse_block, k1=K1, n=N, bq=BQ),
        grid=grid,
        in_specs=[
            pl.BlockSpec((1, BQ, 8), lambda bi, qi: (bi, qi, 0)),
            pl.BlockSpec((1, 8, N), lambda bi, qi: (bi, 0, 0)),
            pl.BlockSpec((1, N, 8), lambda bi, qi: (bi, 0, 0)),
            pl.BlockSpec((1, N, U), lambda bi, qi: (bi, 0, 0)),
            pl.BlockSpec((8 * K1, U), lambda bi, qi: (0, 0)),
            pl.BlockSpec((8, U), lambda bi, qi: (0, 0)),
            pl.BlockSpec((1, U), lambda bi, qi: (0, 0)),
            pl.BlockSpec((1, U), lambda bi, qi: (0, 0)),
        ],
        out_specs=pl.BlockSpec((1, BQ, K1, 2 * U),
                               lambda bi, qi: (bi, qi, 0, 0)),
        out_shape=jax.ShapeDtypeStruct((B, N, K1, 2 * U), jnp.float32),
    )(pc8, pcT, pc8, feats, wg, a8, v, bb)
    return out


# speedup vs baseline: 12.1075x; 1.1259x over previous
import functools

import jax
import jax.numpy as jnp
from jax import lax
from jax.experimental import pallas as pl
from jax.experimental.pallas import tpu as pltpu
from jax.experimental.pallas import tpu_sc as plsc


def _locse_block(pcq_ref, pcT_ref, pcrows_ref, wg_ref, a8_ref,
                 v_ref, b_ref, r_ref, idx_ref, *, k1, n, bq):
    q = pcq_ref[0]
    pT = pcT_ref[0]
    prow = pcrows_ref[0]
    bi = pl.program_id(0)

    dot = jnp.dot(q, pT, preferred_element_type=jnp.float32)
    sqq = jnp.sum(q * q, axis=1, keepdims=True)
    sqp = jnp.sum(pT * pT, axis=0, keepdims=True)
    d2 = (sqq + sqp) - 2.0 * dot

    iota = jax.lax.broadcasted_iota(jnp.int32, (bq, n), 1)
    inf = jnp.float32(jnp.inf)

    gpts = []
    d2m = d2
    base = bi * n
    bigi = jnp.int32(n)
    for k in range(k1):
        minv = jnp.min(d2m, axis=1, keepdims=True)
        idx = jnp.min(jnp.where(d2m == minv, iota, bigi), axis=1,
                      keepdims=True)
        onehot = (iota == idx)
        d2m = jnp.where(onehot, inf, d2m)
        oh_f = onehot.astype(jnp.float32)
        p_k = jnp.dot(oh_f, prow, preferred_element_type=jnp.float32)
        gpts.append(p_k)
        idx_ref[0, :, k] = idx[:, 0] + base

    g = jnp.concatenate(gpts, axis=1)
    t = jnp.dot(g, wg_ref[...], preferred_element_type=jnp.float32)
    tb = t + b_ref[...]
    for k in range(k1):
        p_k = gpts[k]
        e_k = jnp.dot(p_k, a8_ref[...], preferred_element_type=jnp.float32)
        norm_k = jnp.sqrt(jnp.sum(p_k * p_k, axis=1, keepdims=True))
        r_ref[0, :, k, :] = jnp.maximum(e_k + norm_k * v_ref[...] + tb, 0.0)


def _make_sc_gather(M, U, n_outer, fire):
    CW = 128
    mesh = plsc.VectorSubcoreMesh(core_axis_name="c", subcore_axis_name="s")
    chunks_per_w = n_outer * fire

    @functools.partial(
        pl.kernel,
        out_type=jax.ShapeDtypeStruct((M, 128), jnp.float32),
        mesh=mesh,
        scratch_types=[
            pltpu.VMEM((chunks_per_w, CW), jnp.int32),
            pltpu.VMEM((fire * CW, 128), jnp.float32),
            pltpu.SemaphoreType.DMA,
        ],
    )
    def sc_gather(feats_hbm, idx_hbm, out_hbm, idx_v, buf, sem):
        wid = lax.axis_index("s") * 2 + lax.axis_index("c")
        cbase = wid * chunks_per_w
        pltpu.sync_copy(idx_hbm.at[wid], idx_v)

        def body(j, carry):
            copies = []
            for i in range(fire):
                copies.append(pltpu.async_copy(
                    feats_hbm.at[idx_v.at[j * fire + i]],
                    buf.at[pl.ds(i * CW, CW)], sem))
            for c in copies:
                c.wait()
            pltpu.sync_copy(
                buf, out_hbm.at[pl.ds((cbase + j * fire) * CW, fire * CW)])
            return carry

        lax.fori_loop(0, n_outer, body, 0)

    return sc_gather


@jax.jit
def kernel(pc, feats, W, b):
    B, N, DIMS = pc.shape
    U = feats.shape[-1]
    K1 = (W.shape[0] - DIMS) // (DIMS + 1)
    BQ = 256
    M = B * N * K1

    pc8 = jnp.pad(pc, ((0, 0), (0, 0), (0, 8 - DIMS)))
    pcT = jnp.transpose(pc8, (0, 2, 1))

    w_xyz = jnp.stack([W[DIMS + (DIMS + 1) * k: 2 * DIMS + (DIMS + 1) * k]
                       for k in range(K1)])
    wg = jnp.pad(w_xyz, ((0, 0), (0, 8 - DIMS), (0, 0))).reshape(8 * K1, U)
    a8 = jnp.pad(W[0:DIMS] - jnp.sum(w_xyz, axis=0), ((0, 8 - DIMS), (0, 0)))
    v = jnp.sum(jnp.stack([W[2 * DIMS + (DIMS + 1) * k] for k in range(K1)]),
                axis=0, keepdims=True)
    bb = b.reshape(1, U)

    grid = (B, N // BQ)
    rhalf, idx3 = pl.pallas_call(
        functools.partial(_locse_block, k1=K1, n=N, bq=BQ),
        grid=grid,
        in_specs=[
            pl.BlockSpec((1, BQ, 8), lambda bi, qi: (bi, qi, 0)),
            pl.BlockSpec((1, 8, N), lambda bi, qi: (bi, 0, 0)),
            pl.BlockSpec((1, N, 8), lambda bi, qi: (bi, 0, 0)),
            pl.BlockSpec((8 * K1, U), lambda bi, qi: (0, 0)),
            pl.BlockSpec((8, U), lambda bi, qi: (0, 0)),
            pl.BlockSpec((1, U), lambda bi, qi: (0, 0)),
            pl.BlockSpec((1, U), lambda bi, qi: (0, 0)),
        ],
        out_specs=[
            pl.BlockSpec((1, BQ, K1, U), lambda bi, qi: (bi, qi, 0, 0)),
            pl.BlockSpec((1, BQ, K1), lambda bi, qi: (bi, qi, 0)),
        ],
        out_shape=[
            jax.ShapeDtypeStruct((B, N, K1, U), jnp.float32),
            jax.ShapeDtypeStruct((B, N, K1), jnp.int32),
        ],
    )(pc8, pcT, pc8, wg, a8, v, bb)

    fire = 4
    n_outer = M // (32 * 128 * fire)
    idx_flat = idx3.reshape(32, n_outer * fire, 128)
    featsf = jnp.pad(feats, ((0, 0), (0, 0), (0, 128 - U))).reshape(B * N, 128)
    fout = _make_sc_gather(M, U, n_outer, fire)(featsf, idx_flat)
    return jnp.concatenate(
        [rhalf, fout.reshape(B, N, K1, 128)[..., :U]], axis=-1)


# speedup vs baseline: 13.4648x; 1.1121x over previous
import functools

import jax
import jax.numpy as jnp
from jax import lax
from jax.experimental import pallas as pl
from jax.experimental.pallas import tpu as pltpu
from jax.experimental.pallas import tpu_sc as plsc


def _extract_block(pcq_ref, pcT_ref, idx_ref, *, k1, n, bq):
    q = pcq_ref[0]
    pT = pcT_ref[0]
    bi = pl.program_id(0)

    dot = jnp.dot(q, pT, preferred_element_type=jnp.float32)
    sqq = jnp.sum(q * q, axis=1, keepdims=True)
    sqp = jnp.sum(pT * pT, axis=0, keepdims=True)
    d2 = (sqq + sqp) - 2.0 * dot

    iota = jax.lax.broadcasted_iota(jnp.int32, (bq, n), 1)
    inf = jnp.float32(jnp.inf)
    bigi = jnp.int32(n)
    base = bi * n
    d2m = d2
    for k in range(k1):
        minv = jnp.min(d2m, axis=1, keepdims=True)
        idx = jnp.min(jnp.where(d2m == minv, iota, bigi), axis=1,
                      keepdims=True)
        d2m = jnp.where(iota == idx, inf, d2m)
        idx_ref[0, :, k] = idx[:, 0] + base


def _encode_block(fr_ref, wg_ref, a8_ref, v_ref, b_ref, out_ref, *, k1):
    fr = fr_ref[0]
    gpts = [fr[:, k, 0:8] for k in range(k1)]
    g = jnp.concatenate(gpts, axis=1)
    t = jnp.dot(g, wg_ref[...], preferred_element_type=jnp.float32)
    tb = t + b_ref[...]
    for k in range(k1):
        p_k = gpts[k]
        e_k = jnp.dot(p_k, a8_ref[...], preferred_element_type=jnp.float32)
        norm_k = jnp.sqrt(jnp.sum(p_k * p_k, axis=1, keepdims=True))
        out_ref[0, :, k, 0:64] = jnp.maximum(
            e_k + norm_k * v_ref[...] + tb, 0.0)
        out_ref[0, :, k, 64:128] = fr[:, k, 64:128]


def _make_sc_gather(M, n_outer, fire):
    CW = 128
    mesh = plsc.VectorSubcoreMesh(core_axis_name="c", subcore_axis_name="s")
    chunks_per_w = n_outer * fire

    @functools.partial(
        pl.kernel,
        out_type=jax.ShapeDtypeStruct((M, 128), jnp.float32),
        mesh=mesh,
        scratch_types=[
            pltpu.VMEM((chunks_per_w, CW), jnp.int32),
            pltpu.VMEM((fire * CW, 128), jnp.float32),
            pltpu.SemaphoreType.DMA,
        ],
    )
    def sc_gather(tbl_hbm, idx_hbm, out_hbm, idx_v, buf, sem):
        wid = lax.axis_index("s") * 2 + lax.axis_index("c")
        cbase = wid * chunks_per_w
        pltpu.sync_copy(idx_hbm.at[wid], idx_v)

        def body(j, carry):
            copies = []
            for i in range(fire):
                copies.append(pltpu.async_copy(
                    tbl_hbm.at[idx_v.at[j * fire + i]],
                    buf.at[pl.ds(i * CW, CW)], sem))
            for c in copies:
                c.wait()
            pltpu.sync_copy(
                buf, out_hbm.at[pl.ds((cbase + j * fire) * CW, fire * CW)])
            return carry

        lax.fori_loop(0, n_outer, body, 0)

    return sc_gather


@jax.jit
def kernel(pc, feats, W, b):
    B, N, DIMS = pc.shape
    U = feats.shape[-1]
    K1 = (W.shape[0] - DIMS) // (DIMS + 1)
    BQ = 256
    M = B * N * K1

    pc8 = jnp.pad(pc, ((0, 0), (0, 0), (0, 8 - DIMS)))
    pcT = jnp.transpose(pc8, (0, 2, 1))

    w_xyz = jnp.stack([W[DIMS + (DIMS + 1) * k: 2 * DIMS + (DIMS + 1) * k]
                       for k in range(K1)])
    wg = jnp.pad(w_xyz, ((0, 0), (0, 8 - DIMS), (0, 0))).reshape(8 * K1, U)
    a8 = jnp.pad(W[0:DIMS] - jnp.sum(w_xyz, axis=0), ((0, 8 - DIMS), (0, 0)))
    v = jnp.sum(jnp.stack([W[2 * DIMS + (DIMS + 1) * k] for k in range(K1)]),
                axis=0, keepdims=True)
    bb = b.reshape(1, U)

    grid = (B, N // BQ)
    idx3 = pl.pallas_call(
        functools.partial(_extract_block, k1=K1, n=N, bq=BQ),
        grid=grid,
        in_specs=[
            pl.BlockSpec((1, BQ, 8), lambda bi, qi: (bi, qi, 0)),
            pl.BlockSpec((1, 8, N), lambda bi, qi: (bi, 0, 0)),
        ],
        out_specs=pl.BlockSpec((1, BQ, K1), lambda bi, qi: (bi, qi, 0)),
        out_shape=jax.ShapeDtypeStruct((B, N, K1), jnp.int32),
    )(pc8, pcT)

    fire = 4
    n_outer = M // (32 * 128 * fire)
    idx_flat = idx3.reshape(32, n_outer * fire, 128)
    tbl = jnp.concatenate(
        [pc8, jnp.zeros((B, N, 64 - 8 - 0), pc.dtype)[..., :56], feats],
        axis=-1).reshape(B * N, 128)
    fout = _make_sc_gather(M, n_outer, fire)(tbl, idx_flat)

    fr4 = fout.reshape(B, N, K1, 128)
    out = pl.pallas_call(
        functools.partial(_encode_block, k1=K1),
        grid=grid,
        in_specs=[
            pl.BlockSpec((1, BQ, K1, 128), lambda bi, qi: (bi, qi, 0, 0)),
            pl.BlockSpec((8 * K1, U), lambda bi, qi: (0, 0)),
            pl.BlockSpec((8, U), lambda bi, qi: (0, 0)),
            pl.BlockSpec((1, U), lambda bi, qi: (0, 0)),
            pl.BlockSpec((1, U), lambda bi, qi: (0, 0)),
        ],
        out_specs=pl.BlockSpec((1, BQ, K1, 128), lambda bi, qi: (bi, qi, 0, 0)),
        out_shape=jax.ShapeDtypeStruct((B, N, K1, 2 * U), jnp.float32),
    )(fr4, wg, a8, v, bb)
    return out
